# Initial kernel scaffold; baseline (speedup 1.0000x reference)
import jax
import jax.numpy as jnp
from jax.experimental import pallas as pl


def kernel(embedding, kgg_table, rel_table, scg_ids, relation_ids, kgg_ids):
    raise NotImplementedError("write your pallas kernel here")


# speedup vs baseline: 2.6423x; 2.6423x over previous
import functools

import jax
import jax.numpy as jnp
from jax import lax
from jax.experimental import pallas as pl
from jax.experimental.pallas import tpu as pltpu
from jax.experimental.pallas import tpu_sc as plsc


def _gather3(B, D, NC, NS):
    NW = NC * NS
    b_per_w = B // NW
    mesh = plsc.VectorSubcoreMesh(core_axis_name="c", subcore_axis_name="s")

    @functools.partial(
        pl.kernel,
        mesh=mesh,
        out_type=(
            jax.ShapeDtypeStruct((B, D), jnp.float32),
            jax.ShapeDtypeStruct((B, D), jnp.float32),
            jax.ShapeDtypeStruct((B, D), jnp.float32),
        ),
        scratch_types=[
            pltpu.VMEM((b_per_w,), jnp.int32),
            pltpu.VMEM((b_per_w, D), jnp.float32),
            pltpu.SemaphoreType.DMA,
        ],
    )
    def k(emb_hbm, kgg_hbm, rel_hbm, scg_ids_hbm, kgg_ids_hbm, rel_ids_hbm,
          out_scg, out_kgg, out_rel, idx_v, rows_v, sem):
        wid = lax.axis_index("s") * NC + lax.axis_index("c")
        base = wid * b_per_w
        for ids_hbm, table_hbm, out_hbm in (
            (scg_ids_hbm, emb_hbm, out_scg),
            (kgg_ids_hbm, kgg_hbm, out_kgg),
            (rel_ids_hbm, rel_hbm, out_rel),
        ):
            pltpu.sync_copy(ids_hbm.at[pl.ds(base, b_per_w)], idx_v)
            pltpu.async_copy(table_hbm.at[idx_v], rows_v, sem).wait()
            pltpu.sync_copy(rows_v, out_hbm.at[pl.ds(base, b_per_w)])

    return k


def kernel(embedding, kgg_table, rel_table, scg_ids, relation_ids, kgg_ids):
    B = scg_ids.shape[0]
    D = embedding.shape[1]
    info = plsc.get_sparse_core_info()
    NC, NS = info.num_cores, info.num_subcores
    k = _gather3(B, D, NC, NS)
    scg, kgg, rel = k(
        embedding,
        kgg_table,
        rel_table,
        scg_ids.astype(jnp.int32),
        kgg_ids.astype(jnp.int32),
        relation_ids.astype(jnp.int32),
    )
    return (scg, kgg, rel)
